# Initial kernel scaffold; baseline (speedup 1.0000x reference)
import jax
import jax.numpy as jnp
from jax.experimental import pallas as pl


def kernel(features, edge_index, Ws1, bs1, Wn1, bn1, Ws2, bs2, Wn2, bn2):
    raise NotImplementedError("write your pallas kernel here")


# speedup vs baseline: 6.8994x; 6.8994x over previous
import functools

import jax
import jax.numpy as jnp
from jax import lax
from jax.experimental import pallas as pl
from jax.experimental.pallas import tpu as pltpu
from jax.experimental.pallas import tpu_sc as plsc

NC = 2
NS = 16
NW = NC * NS
CH = 80


def _mesh():
    return plsc.VectorSubcoreMesh(core_axis_name="c", subcore_axis_name="s",
                                  num_cores=NC, num_subcores=NS)


def _worker_ids():
    c = lax.axis_index("c")
    s = lax.axis_index("s")
    return c, s, s * NC + c


def _agg_body(rows_per_tile, nch, epw, table, src1, dst1, zeros_h, part,
              idx_s0, idx_d0, idx_s1, idx_d1, rows0, rows1, acc_sh,
              sem0, sem1):
    c, s, wid = _worker_ids()
    r0 = s * rows_per_tile
    pltpu.sync_copy(zeros_h.at[pl.ds(r0, rows_per_tile)],
                    acc_sh.at[pl.ds(r0, rows_per_tile)])
    plsc.subcore_barrier()

    base0 = wid * jnp.int32(epw)
    pltpu.sync_copy(src1.at[pl.ds(base0, CH)], idx_s0)
    pltpu.sync_copy(dst1.at[pl.ds(base0, CH)], idx_d0)
    pltpu.async_copy(table.at[idx_s0], rows0, sem0)

    def pair(t, carry):
        j0 = 2 * t
        b1 = base0 + (j0 + 1) * jnp.int32(CH)
        pltpu.sync_copy(src1.at[pl.ds(b1, CH)], idx_s1)
        pltpu.sync_copy(dst1.at[pl.ds(b1, CH)], idx_d1)
        pltpu.async_copy(table.at[idx_s1], rows1, sem1)
        pltpu.make_async_copy(table.at[idx_s0], rows0, sem0).wait()
        pltpu.sync_copy(rows0, acc_sh.at[idx_d0], add=True)
        b2 = base0 + (j0 + 2) * jnp.int32(CH)
        pltpu.sync_copy(src1.at[pl.ds(b2, CH)], idx_s0)
        pltpu.sync_copy(dst1.at[pl.ds(b2, CH)], idx_d0)
        pltpu.async_copy(table.at[idx_s0], rows0, sem0)
        pltpu.make_async_copy(table.at[idx_s1], rows1, sem1).wait()
        pltpu.sync_copy(rows1, acc_sh.at[idx_d1], add=True)
        return carry

    lax.fori_loop(jnp.int32(0), jnp.int32((nch - 1) // 2), pair, jnp.int32(0))
    pltpu.make_async_copy(table.at[idx_s0], rows0, sem0).wait()
    pltpu.sync_copy(rows0, acc_sh.at[idx_d0], add=True)
    plsc.subcore_barrier()
    pltpu.sync_copy(acc_sh.at[pl.ds(r0, rows_per_tile)],
                    part.at[c, pl.ds(r0, rows_per_tile)])


def _make_agg(n, d, nch, epw):
    assert nch % 2 == 1
    rows_per_tile = n // NS
    out_type = [jax.ShapeDtypeStruct((NC, n, d), jnp.float32)]
    scratch = [
        pltpu.VMEM((CH,), jnp.int32),
        pltpu.VMEM((CH,), jnp.int32),
        pltpu.VMEM((CH,), jnp.int32),
        pltpu.VMEM((CH,), jnp.int32),
        pltpu.VMEM((CH, d), jnp.float32),
        pltpu.VMEM((CH, d), jnp.float32),
        pltpu.VMEM_SHARED((n, d), jnp.float32),
        pltpu.SemaphoreType.DMA,
        pltpu.SemaphoreType.DMA,
    ]
    body = functools.partial(_agg_body, rows_per_tile, nch, epw)
    return pl.kernel(body, mesh=_mesh(), out_type=out_type,
                     scratch_types=scratch)


def _deg_body(rows_per_tile, nch, epw, dst1, zeros_h, ones_h, degp,
              idx_d0, idx_d1, ones_v, deg_sh, sem0, sem1):
    c, s, wid = _worker_ids()
    r0 = s * rows_per_tile
    pltpu.sync_copy(zeros_h.at[pl.ds(r0, rows_per_tile)],
                    deg_sh.at[pl.ds(r0, rows_per_tile)])
    pltpu.sync_copy(ones_h, ones_v)
    plsc.subcore_barrier()

    base0 = wid * jnp.int32(epw)
    pltpu.sync_copy(dst1.at[pl.ds(base0, CH)], idx_d0)
    pltpu.async_copy(ones_v, deg_sh.at[idx_d0], sem0, add=True)

    def pair(t, carry):
        b1 = base0 + (2 * t + 1) * jnp.int32(CH)
        pltpu.sync_copy(dst1.at[pl.ds(b1, CH)], idx_d1)
        pltpu.async_copy(ones_v, deg_sh.at[idx_d1], sem1, add=True)
        pltpu.make_async_copy(ones_v, deg_sh.at[idx_d0], sem0).wait()
        b2 = base0 + (2 * t + 2) * jnp.int32(CH)
        pltpu.sync_copy(dst1.at[pl.ds(b2, CH)], idx_d0)
        pltpu.async_copy(ones_v, deg_sh.at[idx_d0], sem0, add=True)
        pltpu.make_async_copy(ones_v, deg_sh.at[idx_d1], sem1).wait()
        return carry

    lax.fori_loop(jnp.int32(0), jnp.int32((nch - 1) // 2), pair, jnp.int32(0))
    pltpu.make_async_copy(ones_v, deg_sh.at[idx_d0], sem0).wait()
    plsc.subcore_barrier()
    pltpu.sync_copy(deg_sh.at[pl.ds(r0, rows_per_tile)],
                    degp.at[c, pl.ds(r0, rows_per_tile)])


def _make_deg(n, d, nch, epw):
    assert nch % 2 == 1
    rows_per_tile = n // NS
    out_type = [jax.ShapeDtypeStruct((NC, n, d), jnp.float32)]
    scratch = [
        pltpu.VMEM((CH,), jnp.int32),
        pltpu.VMEM((CH,), jnp.int32),
        pltpu.VMEM((CH, d), jnp.float32),
        pltpu.VMEM_SHARED((n, d), jnp.float32),
        pltpu.SemaphoreType.DMA,
        pltpu.SemaphoreType.DMA,
    ]
    body = functools.partial(_deg_body, rows_per_tile, nch, epw)
    return pl.kernel(body, mesh=_mesh(), out_type=out_type,
                     scratch_types=scratch)


def _gather_body(epw, nch, table, src1, dst1, out_s, out_d,
                 idx_s0, idx_d0, idx_s1, idx_d1, rows_a, rows_b,
                 sem_a, sem_b):
    c, s, wid = _worker_ids()
    base0 = wid * jnp.int32(epw)
    pltpu.sync_copy(src1.at[pl.ds(base0, CH)], idx_s0)
    pltpu.sync_copy(dst1.at[pl.ds(base0, CH)], idx_d0)
    pltpu.async_copy(table.at[idx_s0], rows_a, sem_a)

    def pair(t, carry):
        j0 = 2 * t
        b0 = base0 + j0 * jnp.int32(CH)
        b1 = b0 + jnp.int32(CH)
        b2 = b1 + jnp.int32(CH)
        pltpu.async_copy(table.at[idx_d0], rows_b, sem_b)
        pltpu.sync_copy(src1.at[pl.ds(b1, CH)], idx_s1)
        pltpu.sync_copy(dst1.at[pl.ds(b1, CH)], idx_d1)
        pltpu.make_async_copy(table.at[idx_s0], rows_a, sem_a).wait()
        pltpu.sync_copy(rows_a, out_s.at[pl.ds(b0, CH)])
        pltpu.async_copy(table.at[idx_s1], rows_a, sem_a)
        pltpu.make_async_copy(table.at[idx_d0], rows_b, sem_b).wait()
        pltpu.sync_copy(rows_b, out_d.at[pl.ds(b0, CH)])
        pltpu.async_copy(table.at[idx_d1], rows_b, sem_b)
        pltpu.sync_copy(src1.at[pl.ds(b2, CH)], idx_s0)
        pltpu.sync_copy(dst1.at[pl.ds(b2, CH)], idx_d0)
        pltpu.make_async_copy(table.at[idx_s1], rows_a, sem_a).wait()
        pltpu.sync_copy(rows_a, out_s.at[pl.ds(b1, CH)])
        pltpu.async_copy(table.at[idx_s0], rows_a, sem_a)
        pltpu.make_async_copy(table.at[idx_d1], rows_b, sem_b).wait()
        pltpu.sync_copy(rows_b, out_d.at[pl.ds(b1, CH)])
        return carry

    lax.fori_loop(jnp.int32(0), jnp.int32((nch - 1) // 2), pair, jnp.int32(0))
    blast = base0 + (nch - 1) * jnp.int32(CH)
    pltpu.async_copy(table.at[idx_d0], rows_b, sem_b)
    pltpu.make_async_copy(table.at[idx_s0], rows_a, sem_a).wait()
    pltpu.sync_copy(rows_a, out_s.at[pl.ds(blast, CH)])
    pltpu.make_async_copy(table.at[idx_d0], rows_b, sem_b).wait()
    pltpu.sync_copy(rows_b, out_d.at[pl.ds(blast, CH)])


def _make_gather2(e, d, nch):
    assert nch % 2 == 1
    epw = e // NW
    out_type = [jax.ShapeDtypeStruct((e, d), jnp.float32),
                jax.ShapeDtypeStruct((e, d), jnp.float32)]
    scratch = [
        pltpu.VMEM((CH,), jnp.int32),
        pltpu.VMEM((CH,), jnp.int32),
        pltpu.VMEM((CH,), jnp.int32),
        pltpu.VMEM((CH,), jnp.int32),
        pltpu.VMEM((CH, d), jnp.float32),
        pltpu.VMEM((CH, d), jnp.float32),
        pltpu.SemaphoreType.DMA,
        pltpu.SemaphoreType.DMA,
    ]
    body = functools.partial(_gather_body, epw, nch)
    return pl.kernel(body, mesh=_mesh(), out_type=out_type,
                     scratch_types=scratch)


def _dense_body(relu, x_ref, p0, p1, d0, d1, ws, bs, wn, bn, o_ref):
    deg = jnp.maximum(d0[:, 0:1] + d1[:, 0:1], 1.0)
    hn = (p0[...] + p1[...]) / deg
    acc = jnp.dot(x_ref[...], ws[...], preferred_element_type=jnp.float32)
    acc = acc + jnp.dot(hn, wn[...], preferred_element_type=jnp.float32)
    acc = acc + bs[...] + bn[...]
    if relu:
        acc = jnp.maximum(acc, 0.0)
    o_ref[...] = acc


def _dense(relu, n, d, h, x, p0, p1, d0, d1, ws, bs, wn, bn):
    blk = next(b for b in (1000, 512, 256, 128, 8) if n % b == 0)
    grid = (n // blk,)
    row = lambda i: (i, jnp.int32(0))
    fixed = lambda i: (jnp.int32(0), jnp.int32(0))
    return pl.pallas_call(
        functools.partial(_dense_body, relu),
        grid=grid,
        in_specs=[
            pl.BlockSpec((blk, d), row),
            pl.BlockSpec((blk, d), row),
            pl.BlockSpec((blk, d), row),
            pl.BlockSpec((blk, d), row),
            pl.BlockSpec((blk, d), row),
            pl.BlockSpec((d, h), fixed),
            pl.BlockSpec((1, h), fixed),
            pl.BlockSpec((d, h), fixed),
            pl.BlockSpec((1, h), fixed),
        ],
        out_specs=pl.BlockSpec((blk, h), row),
        out_shape=jax.ShapeDtypeStruct((n, h), jnp.float32),
    )(x, p0, p1, d0, d1, ws, bs.reshape(1, h), wn, bn.reshape(1, h))


def kernel(features, edge_index, Ws1, bs1, Wn1, bn1, Ws2, bs2, Wn2, bn2):
    features = features.astype(jnp.float32)
    n, d = features.shape
    h = Ws1.shape[1]
    o = Ws2.shape[1]
    e = edge_index.shape[1]
    epw = e // NW
    nch = epw // CH
    npad = -(-n // (NS * 8)) * (NS * 8)

    ei = edge_index.astype(jnp.int32)
    src1 = ei[0]
    dst1 = ei[1]
    z128 = jnp.zeros((npad, d), jnp.float32)
    ones_h = jnp.ones((CH, d), jnp.float32)

    (degp,) = _make_deg(npad, d, nch, epw)(dst1, z128, ones_h)
    (part1,) = _make_agg(npad, d, nch, epw)(features, src1, dst1, z128)
    h1 = _dense(True, n, d, h, features, part1[0, :n], part1[1, :n],
                degp[0, :n], degp[1, :n], Ws1, bs1, Wn1, bn1)
    (part2,) = _make_agg(npad, h, nch, epw)(h1, src1, dst1, z128)
    h2 = _dense(False, n, h, o, h1, part2[0, :n], part2[1, :n],
                degp[0, :n], degp[1, :n], Ws2, bs2, Wn2, bn2)
    src_feat2, dst_feat2 = _make_gather2(e, o, nch)(h2, src1, dst1)
    return (src_feat2, dst_feat2)


# speedup vs baseline: 7.9394x; 1.1507x over previous
import functools

import jax
import jax.numpy as jnp
from jax import lax
from jax.experimental import pallas as pl
from jax.experimental.pallas import tpu as pltpu
from jax.experimental.pallas import tpu_sc as plsc

NC = 2
NS = 16
NW = NC * NS
CH = 80


def _mesh():
    return plsc.VectorSubcoreMesh(core_axis_name="c", subcore_axis_name="s",
                                  num_cores=NC, num_subcores=NS)


def _worker_ids():
    c = lax.axis_index("c")
    s = lax.axis_index("s")
    return c, s, s * NC + c


def _agg_body(rows_per_tile, nch, epw, table, src1, dst1, zeros_h, part,
              *scr):
    idx_s = scr[0:4]
    idx_d = scr[4:8]
    rows = scr[8:12]
    acc_sh = scr[12]
    sem_g = scr[13:17]
    sem_s = scr[17:21]
    c, s, wid = _worker_ids()
    r0 = s * rows_per_tile
    pltpu.sync_copy(zeros_h.at[pl.ds(r0, rows_per_tile)],
                    acc_sh.at[pl.ds(r0, rows_per_tile)])
    plsc.subcore_barrier()

    base0 = wid * jnp.int32(epw)
    for x in (0, 1):
        b = base0 + x * jnp.int32(CH)
        pltpu.sync_copy(src1.at[pl.ds(b, CH)], idx_s[x])
        pltpu.sync_copy(dst1.at[pl.ds(b, CH)], idx_d[x])
        pltpu.async_copy(table.at[idx_s[x]], rows[x], sem_g[x])

    def group(g, carry):
        j0 = 4 * g
        for x in range(4):
            j = j0 + x
            xn = (x + 2) % 4
            pltpu.make_async_copy(table.at[idx_s[x]], rows[x],
                                  sem_g[x]).wait()
            pltpu.async_copy(rows[x], acc_sh.at[idx_d[x]], sem_s[x],
                             add=True)

            @pl.when(j >= 2)
            def _():
                pltpu.make_async_copy(rows[xn], acc_sh.at[idx_d[xn]],
                                      sem_s[xn]).wait()

            @pl.when(j + 2 < nch)
            def _():
                b = base0 + (j + 2) * jnp.int32(CH)
                pltpu.sync_copy(src1.at[pl.ds(b, CH)], idx_s[xn])
                pltpu.sync_copy(dst1.at[pl.ds(b, CH)], idx_d[xn])
                pltpu.async_copy(table.at[idx_s[xn]], rows[xn], sem_g[xn])
        return carry

    lax.fori_loop(jnp.int32(0), jnp.int32((nch - 1) // 4), group,
                  jnp.int32(0))
    pltpu.make_async_copy(table.at[idx_s[0]], rows[0], sem_g[0]).wait()
    pltpu.async_copy(rows[0], acc_sh.at[idx_d[0]], sem_s[0], add=True)
    for x in (2, 3, 0):
        pltpu.make_async_copy(rows[x], acc_sh.at[idx_d[x]], sem_s[x]).wait()
    plsc.subcore_barrier()
    pltpu.sync_copy(acc_sh.at[pl.ds(r0, rows_per_tile)],
                    part.at[c, pl.ds(r0, rows_per_tile)])


def _make_agg(n, d, nch, epw):
    assert nch % 4 == 1
    rows_per_tile = n // NS
    out_type = [jax.ShapeDtypeStruct((NC, n, d), jnp.float32)]
    scratch = (
        [pltpu.VMEM((CH,), jnp.int32) for _ in range(8)]
        + [pltpu.VMEM((CH, d), jnp.float32) for _ in range(4)]
        + [pltpu.VMEM_SHARED((n, d), jnp.float32)]
        + [pltpu.SemaphoreType.DMA for _ in range(8)]
    )
    body = functools.partial(_agg_body, rows_per_tile, nch, epw)
    return pl.kernel(body, mesh=_mesh(), out_type=out_type,
                     scratch_types=scratch)


def _deg_body(rows_per_tile, nch, epw, dst1, zeros_h, ones_h, degp,
              idx_d0, idx_d1, ones_v, deg_sh, sem0, sem1):
    c, s, wid = _worker_ids()
    r0 = s * rows_per_tile
    pltpu.sync_copy(zeros_h.at[pl.ds(r0, rows_per_tile)],
                    deg_sh.at[pl.ds(r0, rows_per_tile)])
    pltpu.sync_copy(ones_h, ones_v)
    plsc.subcore_barrier()

    base0 = wid * jnp.int32(epw)
    pltpu.sync_copy(dst1.at[pl.ds(base0, CH)], idx_d0)
    pltpu.async_copy(ones_v, deg_sh.at[idx_d0], sem0, add=True)

    def pair(t, carry):
        b1 = base0 + (2 * t + 1) * jnp.int32(CH)
        pltpu.sync_copy(dst1.at[pl.ds(b1, CH)], idx_d1)
        pltpu.async_copy(ones_v, deg_sh.at[idx_d1], sem1, add=True)
        pltpu.make_async_copy(ones_v, deg_sh.at[idx_d0], sem0).wait()
        b2 = base0 + (2 * t + 2) * jnp.int32(CH)
        pltpu.sync_copy(dst1.at[pl.ds(b2, CH)], idx_d0)
        pltpu.async_copy(ones_v, deg_sh.at[idx_d0], sem0, add=True)
        pltpu.make_async_copy(ones_v, deg_sh.at[idx_d1], sem1).wait()
        return carry

    lax.fori_loop(jnp.int32(0), jnp.int32((nch - 1) // 2), pair, jnp.int32(0))
    pltpu.make_async_copy(ones_v, deg_sh.at[idx_d0], sem0).wait()
    plsc.subcore_barrier()
    pltpu.sync_copy(deg_sh.at[pl.ds(r0, rows_per_tile)],
                    degp.at[c, pl.ds(r0, rows_per_tile)])


def _make_deg(n, d, nch, epw):
    assert nch % 2 == 1
    rows_per_tile = n // NS
    out_type = [jax.ShapeDtypeStruct((NC, n, d), jnp.float32)]
    scratch = [
        pltpu.VMEM((CH,), jnp.int32),
        pltpu.VMEM((CH,), jnp.int32),
        pltpu.VMEM((CH, d), jnp.float32),
        pltpu.VMEM_SHARED((n, d), jnp.float32),
        pltpu.SemaphoreType.DMA,
        pltpu.SemaphoreType.DMA,
    ]
    body = functools.partial(_deg_body, rows_per_tile, nch, epw)
    return pl.kernel(body, mesh=_mesh(), out_type=out_type,
                     scratch_types=scratch)


def _gather_body(epw, nch, table, src1, dst1, out_s, out_d, *scr):
    idx_s = scr[0:4]
    idx_d = scr[4:8]
    rs = scr[8:12]
    rd = scr[12:16]
    sem_gs = scr[16:20]
    sem_gd = scr[20:24]
    sem_ws = scr[24:28]
    sem_wd = scr[28:32]
    c, s, wid = _worker_ids()
    base0 = wid * jnp.int32(epw)
    for x in (0, 1):
        b = base0 + x * jnp.int32(CH)
        pltpu.sync_copy(src1.at[pl.ds(b, CH)], idx_s[x])
        pltpu.sync_copy(dst1.at[pl.ds(b, CH)], idx_d[x])
        pltpu.async_copy(table.at[idx_s[x]], rs[x], sem_gs[x])
        pltpu.async_copy(table.at[idx_d[x]], rd[x], sem_gd[x])

    def group(g, carry):
        j0 = 4 * g
        for x in range(4):
            j = j0 + x
            xn = (x + 2) % 4
            b = base0 + j * jnp.int32(CH)
            pltpu.make_async_copy(table.at[idx_s[x]], rs[x],
                                  sem_gs[x]).wait()
            pltpu.async_copy(rs[x], out_s.at[pl.ds(b, CH)], sem_ws[x])
            pltpu.make_async_copy(table.at[idx_d[x]], rd[x],
                                  sem_gd[x]).wait()
            pltpu.async_copy(rd[x], out_d.at[pl.ds(b, CH)], sem_wd[x])

            @pl.when(j >= 2)
            def _():
                bp = base0 + (j - 2) * jnp.int32(CH)
                pltpu.make_async_copy(rs[xn], out_s.at[pl.ds(bp, CH)],
                                      sem_ws[xn]).wait()
                pltpu.make_async_copy(rd[xn], out_d.at[pl.ds(bp, CH)],
                                      sem_wd[xn]).wait()

            @pl.when(j + 2 < nch)
            def _():
                bn = base0 + (j + 2) * jnp.int32(CH)
                pltpu.sync_copy(src1.at[pl.ds(bn, CH)], idx_s[xn])
                pltpu.sync_copy(dst1.at[pl.ds(bn, CH)], idx_d[xn])
                pltpu.async_copy(table.at[idx_s[xn]], rs[xn], sem_gs[xn])
                pltpu.async_copy(table.at[idx_d[xn]], rd[xn], sem_gd[xn])
        return carry

    lax.fori_loop(jnp.int32(0), jnp.int32((nch - 1) // 4), group,
                  jnp.int32(0))
    blast = base0 + (nch - 1) * jnp.int32(CH)
    pltpu.make_async_copy(table.at[idx_s[0]], rs[0], sem_gs[0]).wait()
    pltpu.async_copy(rs[0], out_s.at[pl.ds(blast, CH)], sem_ws[0])
    pltpu.make_async_copy(table.at[idx_d[0]], rd[0], sem_gd[0]).wait()
    pltpu.async_copy(rd[0], out_d.at[pl.ds(blast, CH)], sem_wd[0])
    for x in (2, 3, 0):
        bx = base0 + (nch - 3 + ((x - 2) % 4)) * jnp.int32(CH)
        pltpu.make_async_copy(rs[x], out_s.at[pl.ds(bx, CH)],
                              sem_ws[x]).wait()
        pltpu.make_async_copy(rd[x], out_d.at[pl.ds(bx, CH)],
                              sem_wd[x]).wait()


def _make_gather2(e, d, nch):
    assert nch % 4 == 1
    epw = e // NW
    out_type = [jax.ShapeDtypeStruct((e, d), jnp.float32),
                jax.ShapeDtypeStruct((e, d), jnp.float32)]
    scratch = (
        [pltpu.VMEM((CH,), jnp.int32) for _ in range(8)]
        + [pltpu.VMEM((CH, d), jnp.float32) for _ in range(8)]
        + [pltpu.SemaphoreType.DMA for _ in range(16)]
    )
    body = functools.partial(_gather_body, epw, nch)
    return pl.kernel(body, mesh=_mesh(), out_type=out_type,
                     scratch_types=scratch)


def _dense_body(relu, x_ref, p0, p1, d0, d1, ws, bs, wn, bn, o_ref):
    deg = jnp.maximum(d0[:, 0:1] + d1[:, 0:1], 1.0)
    hn = (p0[...] + p1[...]) / deg
    acc = jnp.dot(x_ref[...], ws[...], preferred_element_type=jnp.float32)
    acc = acc + jnp.dot(hn, wn[...], preferred_element_type=jnp.float32)
    acc = acc + bs[...] + bn[...]
    if relu:
        acc = jnp.maximum(acc, 0.0)
    o_ref[...] = acc


def _dense(relu, n, d, h, x, p0, p1, d0, d1, ws, bs, wn, bn):
    blk = next(b for b in (1000, 512, 256, 128, 8) if n % b == 0)
    grid = (n // blk,)
    row = lambda i: (i, jnp.int32(0))
    fixed = lambda i: (jnp.int32(0), jnp.int32(0))
    return pl.pallas_call(
        functools.partial(_dense_body, relu),
        grid=grid,
        in_specs=[
            pl.BlockSpec((blk, d), row),
            pl.BlockSpec((blk, d), row),
            pl.BlockSpec((blk, d), row),
            pl.BlockSpec((blk, d), row),
            pl.BlockSpec((blk, d), row),
            pl.BlockSpec((d, h), fixed),
            pl.BlockSpec((1, h), fixed),
            pl.BlockSpec((d, h), fixed),
            pl.BlockSpec((1, h), fixed),
        ],
        out_specs=pl.BlockSpec((blk, h), row),
        out_shape=jax.ShapeDtypeStruct((n, h), jnp.float32),
    )(x, p0, p1, d0, d1, ws, bs.reshape(1, h), wn, bn.reshape(1, h))


def kernel(features, edge_index, Ws1, bs1, Wn1, bn1, Ws2, bs2, Wn2, bn2):
    features = features.astype(jnp.float32)
    n, d = features.shape
    h = Ws1.shape[1]
    o = Ws2.shape[1]
    e = edge_index.shape[1]
    epw = e // NW
    nch = epw // CH
    npad = -(-n // (NS * 8)) * (NS * 8)

    ei = edge_index.astype(jnp.int32)
    src1 = ei[0]
    dst1 = ei[1]
    z128 = jnp.zeros((npad, d), jnp.float32)
    ones_h = jnp.ones((CH, d), jnp.float32)

    (degp,) = _make_deg(npad, d, nch, epw)(dst1, z128, ones_h)
    (part1,) = _make_agg(npad, d, nch, epw)(features, src1, dst1, z128)
    h1 = _dense(True, n, d, h, features, part1[0, :n], part1[1, :n],
                degp[0, :n], degp[1, :n], Ws1, bs1, Wn1, bn1)
    (part2,) = _make_agg(npad, h, nch, epw)(h1, src1, dst1, z128)
    h2 = _dense(False, n, h, o, h1, part2[0, :n], part2[1, :n],
                degp[0, :n], degp[1, :n], Ws2, bs2, Wn2, bn2)
    src_feat2, dst_feat2 = _make_gather2(e, o, nch)(h2, src1, dst1)
    return (src_feat2, dst_feat2)


# speedup vs baseline: 8.7016x; 1.0960x over previous
import functools

import jax
import jax.numpy as jnp
from jax import lax
from jax.experimental import pallas as pl
from jax.experimental.pallas import tpu as pltpu
from jax.experimental.pallas import tpu_sc as plsc

NC = 2
NS = 16
NW = NC * NS
CH = 80


def _mesh():
    return plsc.VectorSubcoreMesh(core_axis_name="c", subcore_axis_name="s",
                                  num_cores=NC, num_subcores=NS)


def _worker_ids():
    c = lax.axis_index("c")
    s = lax.axis_index("s")
    return c, s, s * NC + c


def _load_chunk_idx(comb1, cb, idx_sd, idx_d):
    pltpu.sync_copy(comb1.at[pl.ds(cb, 2 * CH)], idx_sd)
    for v in range(CH // 16):
        idx_d[pl.ds(v * 16, 16)] = idx_sd[pl.ds(CH + v * 16, 16)]


def _agg_body(rows_per_tile, nch, table, comb1, zeros_h, part,
              *scr):
    idx_sd = scr[0:4]
    idx_d = scr[4:8]
    rows = scr[8:12]
    acc_sh = scr[12]
    sem_g = scr[13:17]
    sem_s = scr[17:21]
    c, s, wid = _worker_ids()
    r0 = s * rows_per_tile
    pltpu.sync_copy(zeros_h.at[pl.ds(r0, rows_per_tile)],
                    acc_sh.at[pl.ds(r0, rows_per_tile)])
    plsc.subcore_barrier()

    cbase0 = wid * jnp.int32(nch * 2 * CH)
    for x in (0, 1):
        cb = cbase0 + x * jnp.int32(2 * CH)
        _load_chunk_idx(comb1, cb, idx_sd[x], idx_d[x])
        pltpu.async_copy(table.at[idx_sd[x].at[pl.ds(0, CH)]], rows[x],
                         sem_g[x])

    def group(g, carry):
        j0 = 4 * g
        for x in range(4):
            j = j0 + x
            xn = (x + 2) % 4
            pltpu.make_async_copy(table.at[idx_sd[x].at[pl.ds(0, CH)]],
                                  rows[x], sem_g[x]).wait()
            pltpu.async_copy(rows[x], acc_sh.at[idx_d[x]], sem_s[x],
                             add=True)

            @pl.when(j >= 2)
            def _():
                pltpu.make_async_copy(rows[xn], acc_sh.at[idx_d[xn]],
                                      sem_s[xn]).wait()

            @pl.when(j + 2 < nch)
            def _():
                cb = cbase0 + (j + 2) * jnp.int32(2 * CH)
                _load_chunk_idx(comb1, cb, idx_sd[xn], idx_d[xn])
                pltpu.async_copy(table.at[idx_sd[xn].at[pl.ds(0, CH)]],
                                 rows[xn], sem_g[xn])
        return carry

    lax.fori_loop(jnp.int32(0), jnp.int32((nch - 1) // 4), group,
                  jnp.int32(0))
    pltpu.make_async_copy(table.at[idx_sd[0].at[pl.ds(0, CH)]], rows[0],
                          sem_g[0]).wait()
    pltpu.async_copy(rows[0], acc_sh.at[idx_d[0]], sem_s[0], add=True)
    for x in (2, 3, 0):
        pltpu.make_async_copy(rows[x], acc_sh.at[idx_d[x]], sem_s[x]).wait()
    plsc.subcore_barrier()
    pltpu.sync_copy(acc_sh.at[pl.ds(r0, rows_per_tile)],
                    part.at[c, pl.ds(r0, rows_per_tile)])


def _make_agg(n, d, nch):
    assert nch % 4 == 1
    rows_per_tile = n // NS
    out_type = [jax.ShapeDtypeStruct((NC, n, d), jnp.float32)]
    scratch = (
        [pltpu.VMEM((2 * CH,), jnp.int32) for _ in range(4)]
        + [pltpu.VMEM((CH,), jnp.int32) for _ in range(4)]
        + [pltpu.VMEM((CH, d), jnp.float32) for _ in range(4)]
        + [pltpu.VMEM_SHARED((n, d), jnp.float32)]
        + [pltpu.SemaphoreType.DMA for _ in range(8)]
    )
    body = functools.partial(_agg_body, rows_per_tile, nch)
    return pl.kernel(body, mesh=_mesh(), out_type=out_type,
                     scratch_types=scratch)


def _deg_body(rows_per_tile, nch, epw, dst1, zeros_h, ones_h, degp,
              idx_d0, idx_d1, ones_v, deg_sh, sem0, sem1):
    c, s, wid = _worker_ids()
    r0 = s * rows_per_tile
    pltpu.sync_copy(zeros_h.at[pl.ds(r0, rows_per_tile)],
                    deg_sh.at[pl.ds(r0, rows_per_tile)])
    pltpu.sync_copy(ones_h, ones_v)
    plsc.subcore_barrier()

    base0 = wid * jnp.int32(epw)
    pltpu.sync_copy(dst1.at[pl.ds(base0, CH)], idx_d0)
    pltpu.async_copy(ones_v, deg_sh.at[idx_d0], sem0, add=True)

    def pair(t, carry):
        b1 = base0 + (2 * t + 1) * jnp.int32(CH)
        pltpu.sync_copy(dst1.at[pl.ds(b1, CH)], idx_d1)
        pltpu.async_copy(ones_v, deg_sh.at[idx_d1], sem1, add=True)
        pltpu.make_async_copy(ones_v, deg_sh.at[idx_d0], sem0).wait()
        b2 = base0 + (2 * t + 2) * jnp.int32(CH)
        pltpu.sync_copy(dst1.at[pl.ds(b2, CH)], idx_d0)
        pltpu.async_copy(ones_v, deg_sh.at[idx_d0], sem0, add=True)
        pltpu.make_async_copy(ones_v, deg_sh.at[idx_d1], sem1).wait()
        return carry

    lax.fori_loop(jnp.int32(0), jnp.int32((nch - 1) // 2), pair, jnp.int32(0))
    pltpu.make_async_copy(ones_v, deg_sh.at[idx_d0], sem0).wait()
    plsc.subcore_barrier()
    pltpu.sync_copy(deg_sh.at[pl.ds(r0, rows_per_tile)],
                    degp.at[c, pl.ds(r0, rows_per_tile)])


def _make_deg(n, d, nch, epw):
    assert nch % 2 == 1
    rows_per_tile = n // NS
    out_type = [jax.ShapeDtypeStruct((NC, n, d), jnp.float32)]
    scratch = [
        pltpu.VMEM((CH,), jnp.int32),
        pltpu.VMEM((CH,), jnp.int32),
        pltpu.VMEM((CH, d), jnp.float32),
        pltpu.VMEM_SHARED((n, d), jnp.float32),
        pltpu.SemaphoreType.DMA,
        pltpu.SemaphoreType.DMA,
    ]
    body = functools.partial(_deg_body, rows_per_tile, nch, epw)
    return pl.kernel(body, mesh=_mesh(), out_type=out_type,
                     scratch_types=scratch)


def _gather_body(epw, nch, table, comb1, out_s, out_d, *scr):
    idx_sd = scr[0:4]
    rs = scr[4:8]
    rd = scr[8:12]
    sem_gs = scr[12:16]
    sem_gd = scr[16:20]
    sem_ws = scr[20:24]
    sem_wd = scr[24:28]
    c, s, wid = _worker_ids()
    base0 = wid * jnp.int32(epw)
    cbase0 = wid * jnp.int32(nch * 2 * CH)
    for x in (0, 1):
        cb = cbase0 + x * jnp.int32(2 * CH)
        pltpu.sync_copy(comb1.at[pl.ds(cb, 2 * CH)], idx_sd[x])
        pltpu.async_copy(table.at[idx_sd[x].at[pl.ds(0, CH)]], rs[x],
                         sem_gs[x])
        pltpu.async_copy(table.at[idx_sd[x].at[pl.ds(CH, CH)]], rd[x],
                         sem_gd[x])

    def group(g, carry):
        j0 = 4 * g
        for x in range(4):
            j = j0 + x
            xn = (x + 2) % 4
            b = base0 + j * jnp.int32(CH)
            pltpu.make_async_copy(table.at[idx_sd[x].at[pl.ds(0, CH)]],
                                  rs[x], sem_gs[x]).wait()
            pltpu.async_copy(rs[x], out_s.at[pl.ds(b, CH)], sem_ws[x])
            pltpu.make_async_copy(table.at[idx_sd[x].at[pl.ds(CH, CH)]],
                                  rd[x], sem_gd[x]).wait()
            pltpu.async_copy(rd[x], out_d.at[pl.ds(b, CH)], sem_wd[x])

            @pl.when(j >= 2)
            def _():
                bp = base0 + (j - 2) * jnp.int32(CH)
                pltpu.make_async_copy(rs[xn], out_s.at[pl.ds(bp, CH)],
                                      sem_ws[xn]).wait()
                pltpu.make_async_copy(rd[xn], out_d.at[pl.ds(bp, CH)],
                                      sem_wd[xn]).wait()

            @pl.when(j + 2 < nch)
            def _():
                cb = cbase0 + (j + 2) * jnp.int32(2 * CH)
                pltpu.sync_copy(comb1.at[pl.ds(cb, 2 * CH)], idx_sd[xn])
                pltpu.async_copy(table.at[idx_sd[xn].at[pl.ds(0, CH)]],
                                 rs[xn], sem_gs[xn])
                pltpu.async_copy(table.at[idx_sd[xn].at[pl.ds(CH, CH)]],
                                 rd[xn], sem_gd[xn])
        return carry

    lax.fori_loop(jnp.int32(0), jnp.int32((nch - 1) // 4), group,
                  jnp.int32(0))
    blast = base0 + (nch - 1) * jnp.int32(CH)
    pltpu.make_async_copy(table.at[idx_sd[0].at[pl.ds(0, CH)]], rs[0],
                          sem_gs[0]).wait()
    pltpu.async_copy(rs[0], out_s.at[pl.ds(blast, CH)], sem_ws[0])
    pltpu.make_async_copy(table.at[idx_sd[0].at[pl.ds(CH, CH)]], rd[0],
                          sem_gd[0]).wait()
    pltpu.async_copy(rd[0], out_d.at[pl.ds(blast, CH)], sem_wd[0])
    for x in (2, 3, 0):
        bx = base0 + (nch - 3 + ((x - 2) % 4)) * jnp.int32(CH)
        pltpu.make_async_copy(rs[x], out_s.at[pl.ds(bx, CH)],
                              sem_ws[x]).wait()
        pltpu.make_async_copy(rd[x], out_d.at[pl.ds(bx, CH)],
                              sem_wd[x]).wait()


def _make_gather2(e, d, nch):
    assert nch % 4 == 1
    epw = e // NW
    out_type = [jax.ShapeDtypeStruct((e, d), jnp.float32),
                jax.ShapeDtypeStruct((e, d), jnp.float32)]
    scratch = (
        [pltpu.VMEM((2 * CH,), jnp.int32) for _ in range(4)]
        + [pltpu.VMEM((CH, d), jnp.float32) for _ in range(8)]
        + [pltpu.SemaphoreType.DMA for _ in range(16)]
    )
    body = functools.partial(_gather_body, epw, nch)
    return pl.kernel(body, mesh=_mesh(), out_type=out_type,
                     scratch_types=scratch)


def _dense_body(relu, x_ref, p0, p1, d0, d1, ws, bs, wn, bn, o_ref):
    deg = jnp.maximum(d0[:, 0:1] + d1[:, 0:1], 1.0)
    hn = (p0[...] + p1[...]) / deg
    acc = jnp.dot(x_ref[...], ws[...], preferred_element_type=jnp.float32)
    acc = acc + jnp.dot(hn, wn[...], preferred_element_type=jnp.float32)
    acc = acc + bs[...] + bn[...]
    if relu:
        acc = jnp.maximum(acc, 0.0)
    o_ref[...] = acc


def _dense(relu, n, d, h, x, p0, p1, d0, d1, ws, bs, wn, bn):
    blk = next(b for b in (1000, 512, 256, 128, 8) if n % b == 0)
    grid = (n // blk,)
    row = lambda i: (i, jnp.int32(0))
    fixed = lambda i: (jnp.int32(0), jnp.int32(0))
    return pl.pallas_call(
        functools.partial(_dense_body, relu),
        grid=grid,
        in_specs=[
            pl.BlockSpec((blk, d), row),
            pl.BlockSpec((blk, d), row),
            pl.BlockSpec((blk, d), row),
            pl.BlockSpec((blk, d), row),
            pl.BlockSpec((blk, d), row),
            pl.BlockSpec((d, h), fixed),
            pl.BlockSpec((1, h), fixed),
            pl.BlockSpec((d, h), fixed),
            pl.BlockSpec((1, h), fixed),
        ],
        out_specs=pl.BlockSpec((blk, h), row),
        out_shape=jax.ShapeDtypeStruct((n, h), jnp.float32),
    )(x, p0, p1, d0, d1, ws, bs.reshape(1, h), wn, bn.reshape(1, h))


def kernel(features, edge_index, Ws1, bs1, Wn1, bn1, Ws2, bs2, Wn2, bn2):
    features = features.astype(jnp.float32)
    n, d = features.shape
    h = Ws1.shape[1]
    o = Ws2.shape[1]
    e = edge_index.shape[1]
    epw = e // NW
    nch = epw // CH
    npad = -(-n // (NS * 8)) * (NS * 8)

    ei = edge_index.astype(jnp.int32)
    dst1 = ei[1]
    comb1 = jnp.stack([ei[0].reshape(NW, nch, CH),
                       ei[1].reshape(NW, nch, CH)], axis=2).reshape(-1)
    z128 = jnp.zeros((npad, d), jnp.float32)
    ones_h = jnp.ones((CH, d), jnp.float32)

    (degp,) = _make_deg(npad, d, nch, epw)(dst1, z128, ones_h)
    (part1,) = _make_agg(npad, d, nch)(features, comb1, z128)
    h1 = _dense(True, n, d, h, features, part1[0, :n], part1[1, :n],
                degp[0, :n], degp[1, :n], Ws1, bs1, Wn1, bn1)
    (part2,) = _make_agg(npad, h, nch)(h1, comb1, z128)
    h2 = _dense(False, n, h, o, h1, part2[0, :n], part2[1, :n],
                degp[0, :n], degp[1, :n], Ws2, bs2, Wn2, bn2)
    src_feat2, dst_feat2 = _make_gather2(e, o, nch)(h2, comb1)
    return (src_feat2, dst_feat2)
